# baseline (speedup 1.0000x reference)
import functools

import jax
import jax.numpy as jnp
from jax import lax
from jax.experimental import pallas as pl
from jax.experimental.pallas import tpu as pltpu
from jax.experimental.pallas import tpu_sc as plsc

_GN = 8


@functools.lru_cache(maxsize=None)
def _make_sc_kernel(b: int, n: int, d: int, cnt_max: int, base_cnt: int,
                    rem: int, nk: int, kpad: int):
    mesh = plsc.VectorSubcoreMesh(core_axis_name="c", subcore_axis_name="s",
                                  num_cores=2, num_subcores=16)
    nc = mesh.num_cores
    nw = nc * mesh.num_subcores
    nv = d // 16
    rows = b * _GN
    nh_rows = cnt_max * _GN

    @functools.partial(
        pl.kernel,
        out_type=jax.ShapeDtypeStruct((nw, 16), jnp.float32),
        mesh=mesh,
        compiler_params=pltpu.CompilerParams(use_tc_tiling_on_sc=False,
                                             needs_layout_passes=False),
        scratch_types=[
            pltpu.VMEM((nh_rows, kpad), jnp.int32),
            pltpu.VMEM((3, nk, rows), jnp.int32),
            pltpu.VMEM((3, rows, d), jnp.float32),
            pltpu.VMEM((3, nk, rows, d), jnp.float32),
            pltpu.VMEM((16,), jnp.float32),
            pltpu.SemaphoreType.DMA,
            pltpu.SemaphoreType.DMA,
            pltpu.SemaphoreType.DMA,
        ],
    )
    def launch(table, nh8, out, nh_v, idxb, cbuf, nbuf, accv,
               sem0, sem1, sem2):
        wid = lax.axis_index("s") * nc + lax.axis_index("c")
        start = wid * base_cnt + jnp.minimum(wid, rem)
        cnt = base_cnt + (wid < rem).astype(jnp.int32)
        sems = (sem0, sem1, sem2)

        pltpu.sync_copy(nh8.at[pl.ds(start * _GN, nh_rows)], nh_v)
        accv[...] = jnp.zeros((16,), jnp.float32)

        iota = lax.iota(jnp.int32, 16)
        jmod = iota & (_GN - 1)
        boffs = [((iota >> 3) + 2 * h) * n for h in range(rows // 16)]

        def issue(j, p):
            lj = j * _GN
            for k in range(nk):
                col = jnp.full((16,), k + 1, jnp.int32)
                for h in range(rows // 16):
                    vals = plsc.load_gather(nh_v, [lj + jmod, col])
                    idxb[p, k, pl.ds(h * 16, 16)] = vals + boffs[h]
            for bb in range(b):
                pltpu.async_copy(
                    table.at[pl.ds(bb * n + (start + j) * _GN, _GN)],
                    cbuf.at[p, pl.ds(bb * _GN, _GN)], sems[p])
            for k in range(nk):
                pltpu.async_copy(table.at[idxb.at[p, k]], nbuf.at[p, k],
                                 sems[p])

        def wait_chunk(j, p):
            for bb in range(b):
                pltpu.make_async_copy(
                    table.at[pl.ds(bb * n + (start + j) * _GN, _GN)],
                    cbuf.at[p, pl.ds(bb * _GN, _GN)], sems[p]).wait()
            for k in range(nk):
                pltpu.make_async_copy(
                    table.at[idxb.at[p, k]], nbuf.at[p, k], sems[p]).wait()

        def compute(p):
            def row_body(r, a):
                ctr = [cbuf[p, r, pl.ds(16 * v, 16)] for v in range(nv)]
                a = list(a)
                for k in range(nk):
                    for v in range(nv):
                        a[v] = a[v] + jnp.abs(
                            nbuf[p, k, r, pl.ds(16 * v, 16)] - ctr[v])
                return tuple(a)

            zeros = jnp.zeros((16,), jnp.float32)
            accs = lax.fori_loop(0, rows, row_body, (zeros,) * nv)
            tot = accs[0]
            for v in range(1, nv):
                tot = tot + accs[v]
            accv[...] = accv[...] + tot

        issue(0, 0)

        @pl.when(1 < cnt)
        def _():
            issue(1, 1)

        def body3(jj, _):
            j0 = jj * 3
            for p in range(3):
                j = j0 + p

                @pl.when(j + 2 < cnt)
                def _():
                    issue(j + 2, (p + 2) % 3)

                @pl.when(j < cnt)
                def _():
                    wait_chunk(j, p)

            return 0

        lax.fori_loop(0, (cnt + 2) // 3, body3, 0)
        pltpu.sync_copy(accv, out.at[wid])

    return launch, nw


def kernel(output, nh_indices):
    b, n, d = output.shape
    k_all = nh_indices.shape[1]
    nk = k_all - 1
    assert n % _GN == 0
    ngroups = n // _GN

    base_cnt, rem = ngroups // 32, ngroups % 32
    cnt_max = base_cnt + (1 if rem else 0)
    kpad = 8
    launch, nw = _make_sc_kernel(b, n, d, cnt_max, base_cnt, rem, nk, kpad)

    table = output.reshape(b * n, d)
    row_pad = (nw - 1) * base_cnt + min(nw - 1, rem) + cnt_max
    row_pad = row_pad * _GN - n
    nh8 = jnp.pad(nh_indices.astype(jnp.int32),
                  ((0, max(row_pad, 0)), (0, kpad - k_all)))

    partials = launch(table, nh8)
    return jnp.sqrt(jnp.sum(partials) / (b * n * nk * d))


# speedup vs baseline: 1.1440x; 1.1440x over previous
import functools

import jax
import jax.numpy as jnp
from jax import lax
from jax.experimental import pallas as pl
from jax.experimental.pallas import tpu as pltpu
from jax.experimental.pallas import tpu_sc as plsc

_GN = 8
_PB = 50


@functools.lru_cache(maxsize=None)
def _make_pack_kernel(rt: int, d: int):
    mesh = plsc.VectorSubcoreMesh(core_axis_name="c", subcore_axis_name="s",
                                  num_cores=2, num_subcores=16)
    nc = mesh.num_cores
    nw = nc * mesh.num_subcores
    per_w = rt // nw
    nblk = per_w // _PB
    nv = d // 16

    @functools.partial(
        pl.kernel,
        out_type=jax.ShapeDtypeStruct((rt, d // 2), jnp.int32),
        mesh=mesh,
        compiler_params=pltpu.CompilerParams(use_tc_tiling_on_sc=False,
                                             needs_layout_passes=False),
        scratch_types=[
            pltpu.VMEM((2, _PB, d), jnp.float32),
            pltpu.VMEM((2, _PB, d // 2), jnp.int32),
            pltpu.SemaphoreType.DMA,
            pltpu.SemaphoreType.DMA,
            pltpu.SemaphoreType.DMA,
            pltpu.SemaphoreType.DMA,
        ],
    )
    def pack(table, out, fbuf, pbuf, si0, si1, so0, so1):
        wid = lax.axis_index("s") * nc + lax.axis_index("c")
        base = wid * per_w
        sin = (si0, si1)
        sout = (so0, so1)

        def issue_in(j, p):
            pltpu.async_copy(table.at[pl.ds(base + j * _PB, _PB)],
                             fbuf.at[p], sin[p])

        def convert(p):
            def row_body(r, _):
                for g in range(nv // 2):
                    a = fbuf[p, r, pl.ds(32 * g, 16)]
                    bz = fbuf[p, r, pl.ds(32 * g + 16, 16)]
                    w = plsc.pack(a, bz, format=plsc.PackFormat.INTERLEAVED)
                    pbuf[p, r, pl.ds(16 * g, 16)] = plsc.bitcast(w, jnp.int32)
                return 0

            lax.fori_loop(0, _PB, row_body, 0)

        issue_in(0, 0)

        def body(jj, _):
            for p in range(2):
                j = jj * 2 + p

                @pl.when(j + 1 < nblk)
                def _(j=j, p=p):
                    issue_in(j + 1, 1 - p)

                @pl.when(j < nblk)
                def _(j=j, p=p):
                    pltpu.make_async_copy(
                        table.at[pl.ds(base + j * _PB, _PB)],
                        fbuf.at[p], sin[p]).wait()

                    @pl.when(j >= 2)
                    def _():
                        pltpu.make_async_copy(
                            pbuf.at[p],
                            out.at[pl.ds(base + (j - 2) * _PB, _PB)],
                            sout[p]).wait()

                    convert(p)
                    pltpu.async_copy(
                        pbuf.at[p], out.at[pl.ds(base + j * _PB, _PB)],
                        sout[p])
            return 0

        lax.fori_loop(0, (nblk + 1) // 2, body, 0)
        for t in (2, 1):
            j = nblk - t
            pltpu.make_async_copy(
                pbuf.at[j % 2], out.at[pl.ds(base + j * _PB, _PB)],
                sout[j % 2]).wait()

    return pack


@functools.lru_cache(maxsize=None)
def _make_sc_kernel(b: int, n: int, d: int, cnt_max: int, base_cnt: int,
                    rem: int, nk: int, kpad: int):
    mesh = plsc.VectorSubcoreMesh(core_axis_name="c", subcore_axis_name="s",
                                  num_cores=2, num_subcores=16)
    nc = mesh.num_cores
    nw = nc * mesh.num_subcores
    dw = d // 2
    nv = d // 16
    rows = b * _GN
    nh_rows = cnt_max * _GN

    @functools.partial(
        pl.kernel,
        out_type=jax.ShapeDtypeStruct((nw, 16), jnp.float32),
        mesh=mesh,
        compiler_params=pltpu.CompilerParams(use_tc_tiling_on_sc=False,
                                             needs_layout_passes=False),
        scratch_types=[
            pltpu.VMEM((nh_rows, kpad), jnp.int32),
            pltpu.VMEM((3, nk, rows), jnp.int32),
            pltpu.VMEM((3, rows, dw), jnp.int32),
            pltpu.VMEM((3, nk, rows, dw), jnp.int32),
            pltpu.VMEM((16,), jnp.float32),
            pltpu.SemaphoreType.DMA,
            pltpu.SemaphoreType.DMA,
            pltpu.SemaphoreType.DMA,
        ],
    )
    def launch(table, nh8, out, nh_v, idxb, cbuf, nbuf, accv,
               sem0, sem1, sem2):
        wid = lax.axis_index("s") * nc + lax.axis_index("c")
        start = wid * base_cnt + jnp.minimum(wid, rem)
        cnt = base_cnt + (wid < rem).astype(jnp.int32)
        sems = (sem0, sem1, sem2)

        pltpu.sync_copy(nh8.at[pl.ds(start * _GN, nh_rows)], nh_v)
        accv[...] = jnp.zeros((16,), jnp.float32)

        iota = lax.iota(jnp.int32, 16)
        jmod = iota & (_GN - 1)
        boffs = [((iota >> 3) + 2 * h) * n for h in range(rows // 16)]

        def issue(j, p):
            lj = j * _GN
            for k in range(nk):
                col = jnp.full((16,), k + 1, jnp.int32)
                for h in range(rows // 16):
                    vals = plsc.load_gather(nh_v, [lj + jmod, col])
                    idxb[p, k, pl.ds(h * 16, 16)] = vals + boffs[h]
            for bb in range(b):
                pltpu.async_copy(
                    table.at[pl.ds(bb * n + (start + j) * _GN, _GN)],
                    cbuf.at[p, pl.ds(bb * _GN, _GN)], sems[p])
            for k in range(nk):
                pltpu.async_copy(table.at[idxb.at[p, k]], nbuf.at[p, k],
                                 sems[p])

        def wait_chunk(j, p):
            for bb in range(b):
                pltpu.make_async_copy(
                    table.at[pl.ds(bb * n + (start + j) * _GN, _GN)],
                    cbuf.at[p, pl.ds(bb * _GN, _GN)], sems[p]).wait()
            for k in range(nk):
                pltpu.make_async_copy(
                    table.at[idxb.at[p, k]], nbuf.at[p, k], sems[p]).wait()

        def compute(p):
            def row_body(r, a):
                ctr = [plsc.bitcast(cbuf[p, r, pl.ds(16 * g, 16)],
                                    jnp.bfloat16) for g in range(nv // 2)]
                a = list(a)
                for g in range(nv // 2):
                    ds = [jnp.abs(plsc.bitcast(
                              nbuf[p, k, r, pl.ds(16 * g, 16)],
                              jnp.bfloat16) - ctr[g]) for k in range(nk)]
                    while len(ds) > 1:
                        ds = [ds[i] + ds[i + 1]
                              for i in range(0, len(ds) - 1, 2)] + (
                                  [ds[-1]] if len(ds) % 2 else [])
                    lo, hi = plsc.unpack(
                        ds[0], format=plsc.PackFormat.INTERLEAVED)
                    a[2 * g] = a[2 * g] + lo
                    a[2 * g + 1] = a[2 * g + 1] + hi
                return tuple(a)

            zeros = jnp.zeros((16,), jnp.float32)
            accs = lax.fori_loop(0, rows, row_body, (zeros,) * nv)
            tot = accs[0]
            for v in range(1, nv):
                tot = tot + accs[v]
            accv[...] = accv[...] + tot

        issue(0, 0)

        @pl.when(1 < cnt)
        def _():
            issue(1, 1)

        def body3(jj, _):
            j0 = jj * 3
            for p in range(3):
                j = j0 + p

                @pl.when(j + 2 < cnt)
                def _(j=j, p=p):
                    issue(j + 2, (p + 2) % 3)

                @pl.when(j < cnt)
                def _(j=j, p=p):
                    wait_chunk(j, p)
                    compute(p)

            return 0

        lax.fori_loop(0, (cnt + 2) // 3, body3, 0)
        pltpu.sync_copy(accv, out.at[wid])

    return launch


def kernel(output, nh_indices):
    b, n, d = output.shape
    k_all = nh_indices.shape[1]
    nk = k_all - 1
    assert n % _GN == 0
    ngroups = n // _GN
    rt = b * n

    base_cnt, rem = ngroups // 32, ngroups % 32
    cnt_max = base_cnt + (1 if rem else 0)
    kpad = 8
    pack = _make_pack_kernel(rt, d)
    launch = _make_sc_kernel(b, n, d, cnt_max, base_cnt, rem, nk, kpad)
    nw = 32

    table32 = output.reshape(rt, d)
    table = pack(table32)
    row_pad = (nw - 1) * base_cnt + min(nw - 1, rem) + cnt_max
    row_pad = row_pad * _GN - n
    nh8 = jnp.pad(nh_indices.astype(jnp.int32),
                  ((0, max(row_pad, 0)), (0, kpad - k_all)))

    partials = launch(table, nh8)
    return jnp.sqrt(jnp.sum(partials) / (rt * nk * d))


# speedup vs baseline: 1.2023x; 1.0510x over previous
import functools

import jax
import jax.numpy as jnp
from jax import lax
from jax.experimental import pallas as pl
from jax.experimental.pallas import tpu as pltpu
from jax.experimental.pallas import tpu_sc as plsc

_GN = 8
_PB = 50


@functools.lru_cache(maxsize=None)
def _make_pack_kernel(rt: int, d: int):
    mesh = plsc.VectorSubcoreMesh(core_axis_name="c", subcore_axis_name="s",
                                  num_cores=2, num_subcores=16)
    nc = mesh.num_cores
    nw = nc * mesh.num_subcores
    per_w = rt // nw
    nblk = per_w // _PB
    nv = d // 16

    @functools.partial(
        pl.kernel,
        out_type=jax.ShapeDtypeStruct((rt, d // 2), jnp.int32),
        mesh=mesh,
        compiler_params=pltpu.CompilerParams(use_tc_tiling_on_sc=False,
                                             needs_layout_passes=False),
        scratch_types=[
            pltpu.VMEM((3, _PB, d), jnp.float32),
            pltpu.VMEM((3, _PB, d // 2), jnp.int32),
            pltpu.SemaphoreType.DMA,
            pltpu.SemaphoreType.DMA,
            pltpu.SemaphoreType.DMA,
            pltpu.SemaphoreType.DMA,
            pltpu.SemaphoreType.DMA,
            pltpu.SemaphoreType.DMA,
        ],
    )
    def pack(table, out, fbuf, pbuf, si0, si1, si2, so0, so1, so2):
        wid = lax.axis_index("s") * nc + lax.axis_index("c")
        base = wid * per_w
        sin = (si0, si1, si2)
        sout = (so0, so1, so2)

        def issue_in(j, p):
            pltpu.async_copy(table.at[pl.ds(base + j * _PB, _PB)],
                             fbuf.at[p], sin[p])

        def convert(p):
            ur = 5

            def row_body(rr, _):
                for dr in range(ur):
                    r = rr * ur + dr
                    for g in range(nv // 2):
                        a = fbuf[p, r, pl.ds(32 * g, 16)]
                        bz = fbuf[p, r, pl.ds(32 * g + 16, 16)]
                        w = plsc.pack(a, bz,
                                      format=plsc.PackFormat.INTERLEAVED)
                        pbuf[p, r, pl.ds(16 * g, 16)] = plsc.bitcast(
                            w, jnp.int32)
                return 0

            lax.fori_loop(0, _PB // ur, row_body, 0)

        issue_in(0, 0)
        issue_in(1, 1)

        def body(jj, _):
            for p in range(3):
                j = jj * 3 + p

                @pl.when(j + 2 < nblk)
                def _(j=j, p=p):
                    issue_in(j + 2, (p + 2) % 3)

                @pl.when(j < nblk)
                def _(j=j, p=p):
                    pltpu.make_async_copy(
                        table.at[pl.ds(base + j * _PB, _PB)],
                        fbuf.at[p], sin[p]).wait()

                    @pl.when(j >= 3)
                    def _():
                        pltpu.make_async_copy(
                            pbuf.at[p],
                            out.at[pl.ds(base + (j - 3) * _PB, _PB)],
                            sout[p]).wait()

                    convert(p)
                    pltpu.async_copy(
                        pbuf.at[p], out.at[pl.ds(base + j * _PB, _PB)],
                        sout[p])
            return 0

        lax.fori_loop(0, (nblk + 2) // 3, body, 0)
        for t in (3, 2, 1):
            j = nblk - t
            pltpu.make_async_copy(
                pbuf.at[j % 3], out.at[pl.ds(base + j * _PB, _PB)],
                sout[j % 3]).wait()

    return pack


@functools.lru_cache(maxsize=None)
def _make_sc_kernel(b: int, n: int, d: int, cnt_max: int, base_cnt: int,
                    rem: int, nk: int, kpad: int):
    mesh = plsc.VectorSubcoreMesh(core_axis_name="c", subcore_axis_name="s",
                                  num_cores=2, num_subcores=16)
    nc = mesh.num_cores
    nw = nc * mesh.num_subcores
    dw = d // 2
    nv = d // 16
    rows = b * _GN
    nh_rows = cnt_max * _GN

    @functools.partial(
        pl.kernel,
        out_type=jax.ShapeDtypeStruct((nw, 16), jnp.float32),
        mesh=mesh,
        compiler_params=pltpu.CompilerParams(use_tc_tiling_on_sc=False,
                                             needs_layout_passes=False),
        scratch_types=[
            pltpu.VMEM((nh_rows, kpad), jnp.int32),
            pltpu.VMEM((3, nk, rows), jnp.int32),
            pltpu.VMEM((3, rows, dw), jnp.int32),
            pltpu.VMEM((3, nk, rows, dw), jnp.int32),
            pltpu.VMEM((16,), jnp.float32),
            pltpu.SemaphoreType.DMA,
            pltpu.SemaphoreType.DMA,
            pltpu.SemaphoreType.DMA,
        ],
    )
    def launch(table, nh8, out, nh_v, idxb, cbuf, nbuf, accv,
               sem0, sem1, sem2):
        wid = lax.axis_index("s") * nc + lax.axis_index("c")
        start = wid * base_cnt + jnp.minimum(wid, rem)
        cnt = base_cnt + (wid < rem).astype(jnp.int32)
        sems = (sem0, sem1, sem2)

        pltpu.sync_copy(nh8.at[pl.ds(start * _GN, nh_rows)], nh_v)
        accv[...] = jnp.zeros((16,), jnp.float32)

        iota = lax.iota(jnp.int32, 16)
        jmod = iota & (_GN - 1)
        boffs = [((iota >> 3) + 2 * h) * n for h in range(rows // 16)]

        def issue(j, p):
            lj = j * _GN
            for k in range(nk):
                col = jnp.full((16,), k + 1, jnp.int32)
                for h in range(rows // 16):
                    vals = plsc.load_gather(nh_v, [lj + jmod, col])
                    idxb[p, k, pl.ds(h * 16, 16)] = vals + boffs[h]
            for bb in range(b):
                pltpu.async_copy(
                    table.at[pl.ds(bb * n + (start + j) * _GN, _GN)],
                    cbuf.at[p, pl.ds(bb * _GN, _GN)], sems[p])
            for k in range(nk):
                pltpu.async_copy(table.at[idxb.at[p, k]], nbuf.at[p, k],
                                 sems[p])

        def wait_chunk(j, p):
            for bb in range(b):
                pltpu.make_async_copy(
                    table.at[pl.ds(bb * n + (start + j) * _GN, _GN)],
                    cbuf.at[p, pl.ds(bb * _GN, _GN)], sems[p]).wait()
            for k in range(nk):
                pltpu.make_async_copy(
                    table.at[idxb.at[p, k]], nbuf.at[p, k], sems[p]).wait()

        def compute(p):
            def row_body(r, a):
                ctr = [plsc.bitcast(cbuf[p, r, pl.ds(16 * g, 16)],
                                    jnp.bfloat16) for g in range(nv // 2)]
                a = list(a)
                for g in range(nv // 2):
                    ds = [jnp.abs(plsc.bitcast(
                              nbuf[p, k, r, pl.ds(16 * g, 16)],
                              jnp.bfloat16) - ctr[g]) for k in range(nk)]
                    while len(ds) > 1:
                        ds = [ds[i] + ds[i + 1]
                              for i in range(0, len(ds) - 1, 2)] + (
                                  [ds[-1]] if len(ds) % 2 else [])
                    lo, hi = plsc.unpack(
                        ds[0], format=plsc.PackFormat.INTERLEAVED)
                    a[2 * g] = a[2 * g] + lo
                    a[2 * g + 1] = a[2 * g + 1] + hi
                return tuple(a)

            zeros = jnp.zeros((16,), jnp.float32)
            accs = lax.fori_loop(0, rows, row_body, (zeros,) * nv)
            tot = accs[0]
            for v in range(1, nv):
                tot = tot + accs[v]
            accv[...] = accv[...] + tot

        issue(0, 0)

        @pl.when(1 < cnt)
        def _():
            issue(1, 1)

        def body3(jj, _):
            j0 = jj * 3
            for p in range(3):
                j = j0 + p

                @pl.when(j + 2 < cnt)
                def _(j=j, p=p):
                    issue(j + 2, (p + 2) % 3)

                @pl.when(j < cnt)
                def _(j=j, p=p):
                    wait_chunk(j, p)
                    compute(p)

            return 0

        lax.fori_loop(0, (cnt + 2) // 3, body3, 0)
        pltpu.sync_copy(accv, out.at[wid])

    return launch


def kernel(output, nh_indices):
    b, n, d = output.shape
    k_all = nh_indices.shape[1]
    nk = k_all - 1
    assert n % _GN == 0
    ngroups = n // _GN
    rt = b * n

    base_cnt, rem = ngroups // 32, ngroups % 32
    cnt_max = base_cnt + (1 if rem else 0)
    kpad = 8
    pack = _make_pack_kernel(rt, d)
    launch = _make_sc_kernel(b, n, d, cnt_max, base_cnt, rem, nk, kpad)
    nw = 32

    table32 = output.reshape(rt, d)
    table = pack(table32)
    row_pad = (nw - 1) * base_cnt + min(nw - 1, rem) + cnt_max
    row_pad = row_pad * _GN - n
    nh8 = jnp.pad(nh_indices.astype(jnp.int32),
                  ((0, max(row_pad, 0)), (0, kpad - k_all)))

    partials = launch(table, nh8)
    return jnp.sqrt(jnp.sum(partials) / (rt * nk * d))


# speedup vs baseline: 1.2359x; 1.0279x over previous
import functools

import jax
import jax.numpy as jnp
from jax import lax
from jax.experimental import pallas as pl
from jax.experimental.pallas import tpu as pltpu
from jax.experimental.pallas import tpu_sc as plsc

_GN = 16
_PB = 50


@functools.lru_cache(maxsize=None)
def _make_pack_kernel(rt: int, d: int):
    mesh = plsc.VectorSubcoreMesh(core_axis_name="c", subcore_axis_name="s",
                                  num_cores=2, num_subcores=16)
    nc = mesh.num_cores
    nw = nc * mesh.num_subcores
    per_w = rt // nw
    nblk = per_w // _PB
    nv = d // 16

    @functools.partial(
        pl.kernel,
        out_type=jax.ShapeDtypeStruct((rt, d // 2), jnp.int32),
        mesh=mesh,
        compiler_params=pltpu.CompilerParams(use_tc_tiling_on_sc=False,
                                             needs_layout_passes=False),
        scratch_types=[
            pltpu.VMEM((3, _PB, d), jnp.float32),
            pltpu.VMEM((3, _PB, d // 2), jnp.int32),
            pltpu.SemaphoreType.DMA,
            pltpu.SemaphoreType.DMA,
            pltpu.SemaphoreType.DMA,
            pltpu.SemaphoreType.DMA,
            pltpu.SemaphoreType.DMA,
            pltpu.SemaphoreType.DMA,
        ],
    )
    def pack(table, out, fbuf, pbuf, si0, si1, si2, so0, so1, so2):
        wid = lax.axis_index("s") * nc + lax.axis_index("c")
        base = wid * per_w
        sin = (si0, si1, si2)
        sout = (so0, so1, so2)

        def issue_in(j, p):
            pltpu.async_copy(table.at[pl.ds(base + j * _PB, _PB)],
                             fbuf.at[p], sin[p])

        def convert(p):
            ur = 5

            def row_body(rr, _):
                for dr in range(ur):
                    r = rr * ur + dr
                    for g in range(nv // 2):
                        a = fbuf[p, r, pl.ds(32 * g, 16)]
                        bz = fbuf[p, r, pl.ds(32 * g + 16, 16)]
                        w = plsc.pack(a, bz,
                                      format=plsc.PackFormat.INTERLEAVED)
                        pbuf[p, r, pl.ds(16 * g, 16)] = plsc.bitcast(
                            w, jnp.int32)
                return 0

            lax.fori_loop(0, _PB // ur, row_body, 0)

        issue_in(0, 0)
        issue_in(1, 1)

        def body(jj, _):
            for p in range(3):
                j = jj * 3 + p

                @pl.when(j + 2 < nblk)
                def _(j=j, p=p):
                    issue_in(j + 2, (p + 2) % 3)

                @pl.when(j < nblk)
                def _(j=j, p=p):
                    pltpu.make_async_copy(
                        table.at[pl.ds(base + j * _PB, _PB)],
                        fbuf.at[p], sin[p]).wait()

                    @pl.when(j >= 3)
                    def _():
                        pltpu.make_async_copy(
                            pbuf.at[p],
                            out.at[pl.ds(base + (j - 3) * _PB, _PB)],
                            sout[p]).wait()

                    convert(p)
                    pltpu.async_copy(
                        pbuf.at[p], out.at[pl.ds(base + j * _PB, _PB)],
                        sout[p])
            return 0

        lax.fori_loop(0, (nblk + 2) // 3, body, 0)
        for t in (3, 2, 1):
            j = nblk - t
            pltpu.make_async_copy(
                pbuf.at[j % 3], out.at[pl.ds(base + j * _PB, _PB)],
                sout[j % 3]).wait()

    return pack


@functools.lru_cache(maxsize=None)
def _make_sc_kernel(b: int, n: int, d: int, cnt_max: int, base_cnt: int,
                    rem: int, nk: int, kpad: int):
    mesh = plsc.VectorSubcoreMesh(core_axis_name="c", subcore_axis_name="s",
                                  num_cores=2, num_subcores=16)
    nc = mesh.num_cores
    nw = nc * mesh.num_subcores
    dw = d // 2
    nv = d // 16
    rows = b * _GN
    nh_rows = cnt_max * _GN

    @functools.partial(
        pl.kernel,
        out_type=jax.ShapeDtypeStruct((nw, 16), jnp.float32),
        mesh=mesh,
        compiler_params=pltpu.CompilerParams(use_tc_tiling_on_sc=False,
                                             needs_layout_passes=False),
        scratch_types=[
            pltpu.VMEM((nh_rows, kpad), jnp.int32),
            pltpu.VMEM((3, nk, rows), jnp.int32),
            pltpu.VMEM((3, rows, dw), jnp.int32),
            pltpu.VMEM((3, nk, rows, dw), jnp.int32),
            pltpu.VMEM((16,), jnp.float32),
            pltpu.SemaphoreType.DMA,
            pltpu.SemaphoreType.DMA,
            pltpu.SemaphoreType.DMA,
        ],
    )
    def launch(table, nh8, out, nh_v, idxb, cbuf, nbuf, accv,
               sem0, sem1, sem2):
        wid = lax.axis_index("s") * nc + lax.axis_index("c")
        start = wid * base_cnt + jnp.minimum(wid, rem)
        cnt = base_cnt + (wid < rem).astype(jnp.int32)
        sems = (sem0, sem1, sem2)

        pltpu.sync_copy(nh8.at[pl.ds(start * _GN, nh_rows)], nh_v)
        accv[...] = jnp.zeros((16,), jnp.float32)

        iota = lax.iota(jnp.int32, 16)
        sh = _GN.bit_length() - 1
        jmod = iota & (_GN - 1)
        boffs = [((iota >> sh) + (16 * h) // _GN) * n
                 for h in range(rows // 16)]

        def issue(j, p):
            lj = j * _GN
            for k in range(nk):
                col = jnp.full((16,), k + 1, jnp.int32)
                for h in range(rows // 16):
                    vals = plsc.load_gather(nh_v, [lj + jmod, col])
                    idxb[p, k, pl.ds(h * 16, 16)] = vals + boffs[h]
            for bb in range(b):
                pltpu.async_copy(
                    table.at[pl.ds(bb * n + (start + j) * _GN, _GN)],
                    cbuf.at[p, pl.ds(bb * _GN, _GN)], sems[p])
            for k in range(nk):
                pltpu.async_copy(table.at[idxb.at[p, k]], nbuf.at[p, k],
                                 sems[p])

        def wait_chunk(j, p):
            for bb in range(b):
                pltpu.make_async_copy(
                    table.at[pl.ds(bb * n + (start + j) * _GN, _GN)],
                    cbuf.at[p, pl.ds(bb * _GN, _GN)], sems[p]).wait()
            for k in range(nk):
                pltpu.make_async_copy(
                    table.at[idxb.at[p, k]], nbuf.at[p, k], sems[p]).wait()

        def compute(p):
            def row_body(r, a):
                ctr = [plsc.bitcast(cbuf[p, r, pl.ds(16 * g, 16)],
                                    jnp.bfloat16) for g in range(nv // 2)]
                a = list(a)
                for g in range(nv // 2):
                    ds = [jnp.abs(plsc.bitcast(
                              nbuf[p, k, r, pl.ds(16 * g, 16)],
                              jnp.bfloat16) - ctr[g]) for k in range(nk)]
                    while len(ds) > 1:
                        ds = [ds[i] + ds[i + 1]
                              for i in range(0, len(ds) - 1, 2)] + (
                                  [ds[-1]] if len(ds) % 2 else [])
                    lo, hi = plsc.unpack(
                        ds[0], format=plsc.PackFormat.INTERLEAVED)
                    a[2 * g] = a[2 * g] + lo
                    a[2 * g + 1] = a[2 * g + 1] + hi
                return tuple(a)

            zeros = jnp.zeros((16,), jnp.float32)
            accs = lax.fori_loop(0, rows, row_body, (zeros,) * nv)
            tot = accs[0]
            for v in range(1, nv):
                tot = tot + accs[v]
            accv[...] = accv[...] + tot

        issue(0, 0)

        @pl.when(1 < cnt)
        def _():
            issue(1, 1)

        def body3(jj, _):
            j0 = jj * 3
            for p in range(3):
                j = j0 + p

                @pl.when(j + 2 < cnt)
                def _(j=j, p=p):
                    issue(j + 2, (p + 2) % 3)

                @pl.when(j < cnt)
                def _(j=j, p=p):
                    wait_chunk(j, p)
                    compute(p)

            return 0

        lax.fori_loop(0, (cnt + 2) // 3, body3, 0)
        pltpu.sync_copy(accv, out.at[wid])

    return launch


def kernel(output, nh_indices):
    b, n, d = output.shape
    k_all = nh_indices.shape[1]
    nk = k_all - 1
    assert n % _GN == 0
    ngroups = n // _GN
    rt = b * n

    base_cnt, rem = ngroups // 32, ngroups % 32
    cnt_max = base_cnt + (1 if rem else 0)
    kpad = 8
    pack = _make_pack_kernel(rt, d)
    launch = _make_sc_kernel(b, n, d, cnt_max, base_cnt, rem, nk, kpad)
    nw = 32

    table32 = output.reshape(rt, d)
    table = pack(table32)
    row_pad = (nw - 1) * base_cnt + min(nw - 1, rem) + cnt_max
    row_pad = row_pad * _GN - n
    nh8 = jnp.pad(nh_indices.astype(jnp.int32),
                  ((0, max(row_pad, 0)), (0, kpad - k_all)))

    partials = launch(table, nh8)
    return jnp.sqrt(jnp.sum(partials) / (rt * nk * d))
